# baseline (speedup 1.0000x reference)
import functools

import jax
import jax.numpy as jnp
from jax import lax
from jax.experimental import pallas as pl
from jax.experimental.pallas import tpu as pltpu
from jax.experimental.pallas import tpu_sc as plsc

_NBUF = 4
_CHUNK = 800
_LAG = _NBUF - 1


def _gather_fn(n, d, chunk):
    info = plsc.get_sparse_core_info()
    nc, ns = info.num_cores, info.num_subcores
    nw = nc * ns
    per_w = n // nw
    nchunks = per_w // chunk
    nsteps = nchunks // _NBUF
    assert per_w % chunk == 0 and n % nw == 0 and nchunks % _NBUF == 0
    assert nsteps >= 3

    mesh = plsc.VectorSubcoreMesh(core_axis_name="c", subcore_axis_name="s")

    @functools.partial(
        pl.kernel,
        out_type=jax.ShapeDtypeStruct((n, d), jnp.float32),
        mesh=mesh,
        scratch_types=[
            pltpu.VMEM((_NBUF, chunk), jnp.int32),
            pltpu.VMEM((_NBUF, chunk, d), jnp.float32),
            [pltpu.SemaphoreType.DMA] * _NBUF,
            [pltpu.SemaphoreType.DMA] * _NBUF,
            [pltpu.SemaphoreType.DMA] * _NBUF,
        ],
        compiler_params=pltpu.CompilerParams(use_tc_tiling_on_sc=False),
    )
    def run(idx_hbm, table_hbm, out_hbm, idx_v, rows_v, isems, gsems, osems):
        wid = lax.axis_index("s") * nc + lax.axis_index("c")
        base = wid * per_w

        def off(g):
            return pl.multiple_of(base + g * chunk, 8)

        def start_idx(b, g):
            pltpu.async_copy(idx_hbm.at[pl.ds(off(g), chunk)], idx_v.at[b],
                             isems[b])

        def wait_idx(b):
            pltpu.make_async_copy(idx_hbm.at[pl.ds(off(0), chunk)],
                                  idx_v.at[b], isems[b]).wait()

        def start_gather(b):
            pltpu.async_copy(table_hbm.at[idx_v.at[b]], rows_v.at[b], gsems[b])

        def wait_gather(b):
            pltpu.make_async_copy(table_hbm.at[idx_v.at[b]], rows_v.at[b],
                                  gsems[b]).wait()

        def start_out(b, g):
            pltpu.async_copy(rows_v.at[b], out_hbm.at[pl.ds(off(g), chunk)],
                             osems[b])

        def wait_out(b):
            pltpu.make_async_copy(rows_v.at[b],
                                  out_hbm.at[pl.ds(off(0), chunk)],
                                  osems[b]).wait()


        for b in range(_NBUF):
            start_idx(b, b)
        for g in range(_NBUF):
            wait_idx(g)
            start_gather(g)
            if g >= _LAG:
                r = g - _LAG
                br = r % _NBUF
                wait_gather(br)
                start_out(br, r)
                start_idx(br, r + _NBUF)

        def step_body(s, c):
            for b in range(_NBUF):
                g = s * _NBUF + b
                wait_idx(b)
                wait_out(b)
                start_gather(b)
                br = (b - _LAG) % _NBUF
                r = g - _LAG
                wait_gather(br)
                start_out(br, r)
                start_idx(br, r + _NBUF)
            return c

        lax.fori_loop(1, nsteps - 1, step_body, 0)

        for b in range(_NBUF):
            g = (nsteps - 1) * _NBUF + b
            wait_idx(b)
            wait_out(b)
            start_gather(b)
            r = g - _LAG
            br = r % _NBUF
            wait_gather(br)
            start_out(br, r)
            if r + _NBUF < nchunks:
                start_idx(br, r + _NBUF)
        for r in range(nchunks - _LAG, nchunks):
            br = r % _NBUF
            wait_gather(br)
            start_out(br, r)
        for b in range(_NBUF):
            wait_out(b)

    return run


def kernel(input, weight):
    b, h = input.shape
    v, d = weight.shape
    n = b * h
    flat_idx = input.reshape(n).astype(jnp.int32)
    out = _gather_fn(n, d, _CHUNK)(flat_idx, weight)
    return out.reshape(b, h, d)
